# baseline (speedup 1.0000x reference)
import jax
import jax.numpy as jnp
from jax import lax
from jax.experimental import pallas as pl
from jax.experimental.pallas import tpu as pltpu
from jax.experimental.pallas import tpu_sc as plsc

NUM_FIELDS = 26
VOCAB_P1 = 100001
EMBED_DIM = 32
BATCH = 16384
ROW_W = NUM_FIELDS * EMBED_DIM

NC = 2
NS = 16
LANES = 16
NW = NC * NS

CV = 512
NCI = 196
VOCAB_PAD = NCI * CV
LINES_PER_FIELD = VOCAB_PAD // 4
TOT_LINES = NUM_FIELDS * LINES_PER_FIELD
BLK_LINES = CV * EMBED_DIM // 128

XB_PER_W = BATCH // NW
BC = 8
N_BC = XB_PER_W // BC
CROWS = BC * NUM_FIELDS
SUB = 104
N_SUB = CROWS // SUB


def _tc_tr_body(in_ref, out_ref):
    blk = in_ref[0]
    r = blk.reshape(EMBED_DIM, BLK_LINES, 4)
    out_ref[...] = jnp.transpose(r, (1, 2, 0)).reshape(BLK_LINES, 128)


def _wid():
    return lax.axis_index("s") * NC + lax.axis_index("c")


def _gk_body(lines_hbm, x_hbm, out_hbm, xv, idxl, qv, linebuf, rows_v, sem):
    wid = _wid()
    b0 = wid * XB_PER_W
    iota = lax.broadcasted_iota(jnp.int32, (LANES,), 0)
    off_lo = iota * VOCAB_PAD
    off_hi = (iota + 10) * VOCAB_PAD

    def chunk(ci, carry):
        bl0 = ci * BC
        pltpu.sync_copy(x_hbm.at[pl.ds(b0 + bl0, BC), :], xv)

        def prep(b, cc):
            g1 = xv[b, pl.ds(0, LANES)] + off_lo
            g2 = xv[b, pl.ds(10, LANES)] + off_hi
            r = b * NUM_FIELDS
            idxl[pl.ds(r, LANES)] = lax.shift_right_logical(g1, 2)
            idxl[pl.ds(r + 10, LANES)] = lax.shift_right_logical(g2, 2)
            qv[pl.ds(r, LANES)] = lax.bitwise_and(g1, 3)
            qv[pl.ds(r + 10, LANES)] = lax.bitwise_and(g2, 3)
            return cc

        lax.fori_loop(0, BC, prep, 0, unroll=2)

        copies = []
        for j in range(N_SUB):
            copies.append(
                pltpu.async_copy(
                    lines_hbm.at[idxl.at[pl.ds(j * SUB, SUB)]],
                    linebuf.at[pl.ds(j * SUB, SUB)],
                    sem,
                )
            )
        for cp in copies:
            cp.wait()

        def extract(b, cc):
            r = b * NUM_FIELDS
            qa = qv[pl.ds(r, LANES)]
            qb = qv[pl.ds(r + 10, LANES)]
            for f in range(NUM_FIELDS):
                q32 = (qa[f] if f < LANES else qb[f - 10]) * EMBED_DIM
                rows_v[b, pl.ds(f * EMBED_DIM, LANES)] = (
                    linebuf[r + f, pl.ds(q32, LANES)]
                )
                rows_v[b, pl.ds(f * EMBED_DIM + LANES, LANES)] = (
                    linebuf[r + f, pl.ds(q32 + LANES, LANES)]
                )
            return cc

        lax.fori_loop(0, BC, extract, 0, unroll=1)

        pltpu.sync_copy(rows_v, out_hbm.at[pl.ds(b0 + bl0, BC), :])
        return carry

    lax.fori_loop(0, N_BC, chunk, 0)


@jax.jit
def kernel(x, tables):
    tab_t = jnp.swapaxes(tables, 1, 2)

    lines = pl.pallas_call(
        _tc_tr_body,
        grid=(NUM_FIELDS, NCI),
        in_specs=[
            pl.BlockSpec((1, EMBED_DIM, CV), lambda f, ci: (f, 0, ci)),
        ],
        out_specs=pl.BlockSpec(
            (BLK_LINES, 128), lambda f, ci: (f * NCI + ci, 0)
        ),
        out_shape=jax.ShapeDtypeStruct((TOT_LINES, 128), jnp.float32),
    )(tab_t)

    mesh = plsc.VectorSubcoreMesh(core_axis_name="c", subcore_axis_name="s")
    params = pltpu.CompilerParams(
        use_tc_tiling_on_sc=True, needs_layout_passes=False
    )

    out = pl.kernel(
        _gk_body,
        out_type=jax.ShapeDtypeStruct((BATCH, ROW_W), jnp.float32),
        mesh=mesh,
        compiler_params=params,
        scratch_types=[
            pltpu.VMEM((BC, NUM_FIELDS), jnp.int32),
            pltpu.VMEM((CROWS,), jnp.int32),
            pltpu.VMEM((CROWS,), jnp.int32),
            pltpu.VMEM((CROWS, 128), jnp.float32),
            pltpu.VMEM((BC, ROW_W), jnp.float32),
            pltpu.SemaphoreType.DMA,
        ],
    )(lines, x)
    return out


# speedup vs baseline: 3.9370x; 3.9370x over previous
import jax
import jax.numpy as jnp
from jax import lax
from jax.experimental import pallas as pl
from jax.experimental.pallas import tpu as pltpu
from jax.experimental.pallas import tpu_sc as plsc

NUM_FIELDS = 26
VOCAB_P1 = 100001
EMBED_DIM = 32
BATCH = 16384
ROW_W = NUM_FIELDS * EMBED_DIM

NC = 2
NS = 16
LANES = 16
NW = NC * NS

VOCAB_PAD = 100032
LINES_PER_FIELD = VOCAB_PAD // 4
TOT_LINES = NUM_FIELDS * LINES_PER_FIELD

CV = 512
FULL_CHUNKS = VOCAB_P1 // CV
CV2 = 128
N_FULL = NUM_FIELDS * FULL_CHUNKS
K_MAX = (N_FULL + NW - 1) // NW
T_STEPS = (K_MAX + 2 + 1) // 2
BLK_LINES = CV * EMBED_DIM // 128
BLK2_LINES = CV2 * EMBED_DIM // 128
TAIL33 = VOCAB_P1 - FULL_CHUNKS * CV - CV2
TAIL33_WORDS = NUM_FIELDS * TAIL33 * EMBED_DIM
TAIL_LINES = (VOCAB_PAD - FULL_CHUNKS * CV - CV2) * EMBED_DIM // 128

XB_PER_W = BATCH // NW
BC = 8
N_BC = XB_PER_W // BC
CROWS = BC * NUM_FIELDS
SUB = 104
N_SUB = CROWS // SUB


def _wid():
    return lax.axis_index("s") * NC + lax.axis_index("c")


def _tr_body(tab_t, tail33, lines_out, in0, in1, out0, out1, tbuf,
             si0, si1, so0, so1):
    wid = _wid()
    iota = lax.broadcasted_iota(jnp.int32, (LANES,), 0)

    def src_of(item):
        f = item // FULL_CHUNKS
        ci = item % FULL_CHUNKS
        return f, ci, tab_t.at[f, :, pl.ds(ci * CV, CV)]

    def dst_of(item):
        f = item // FULL_CHUNKS
        ci = item % FULL_CHUNKS
        line0 = f * LINES_PER_FIELD + ci * BLK_LINES
        return lines_out.at[pl.ds(line0, BLK_LINES), :]

    rowpat = lax.shift_right_logical(iota, 2)
    colpat = lax.bitwise_and(iota, 3) * EMBED_DIM

    def transpose_block(inb, outb, n_v):
        def tr(vv, cc):
            rowv = rowpat + vv * 4
            for d in range(EMBED_DIM):
                val = inb[d, pl.ds(vv * LANES, LANES)]
                plsc.store_scatter(outb, [rowv, colpat + d], val)
            return cc

        lax.fori_loop(0, n_v // LANES, tr, 0, unroll=2)

    bufs = ((in0, out0, si0, so0), (in1, out1, si1, so1))

    for par in range(2):
        item = par * NW + wid

        @pl.when(item < N_FULL)
        def _(par=par, item=item):
            inb, _, sin, _ = bufs[par]
            pltpu.async_copy(src_of(item)[2], inb.at[:, pl.ds(0, CV)], sin)

    def step(t, c):
        for par in range(2):
            k = t * 2 + par
            item = k * NW + wid
            inb, outb, sin, sout = bufs[par]

            @pl.when((k >= 2) & ((k - 2) * NW + wid < N_FULL))
            def _():
                pltpu.make_async_copy(outb.at[:, pl.ds(0, 128)], dst_of((k - 2) * NW + wid), sout).wait()

            @pl.when(item < N_FULL)
            def _():
                pltpu.make_async_copy(src_of(item)[2], inb.at[:, pl.ds(0, CV)], sin).wait()
                transpose_block(inb, outb, CV)
                pltpu.async_copy(outb.at[:, pl.ds(0, 128)], dst_of(item), sout)

                nxt = (k + 2) * NW + wid

                @pl.when(nxt < N_FULL)
                def _():
                    pltpu.async_copy(src_of(nxt)[2], inb.at[:, pl.ds(0, CV)], sin)

        return c

    lax.fori_loop(0, T_STEPS, step, 0)

    @pl.when(wid < NUM_FIELDS)
    def _():
        f = wid
        pltpu.sync_copy(
            tab_t.at[f, :, pl.ds(FULL_CHUNKS * CV, CV2)],
            in0.at[:, pl.ds(0, CV2)],
        )
        transpose_block(in0, out0, CV2)
        line0 = f * LINES_PER_FIELD + FULL_CHUNKS * BLK_LINES
        pltpu.sync_copy(
            out0.at[pl.ds(0, BLK2_LINES), pl.ds(0, 128)],
            lines_out.at[pl.ds(line0, BLK2_LINES), :],
        )

        pltpu.sync_copy(
            tail33.at[pl.ds(f * TAIL33 * EMBED_DIM, TAIL33 * EMBED_DIM)], tbuf
        )
        for l in range(TAIL_LINES):
            for qq in range(4):
                v = l * 4 + qq
                if v < TAIL33:
                    out0[l, pl.ds(qq * EMBED_DIM, LANES)] = (
                        tbuf[pl.ds(v * EMBED_DIM, LANES)]
                    )
                    out0[l, pl.ds(qq * EMBED_DIM + LANES, LANES)] = (
                        tbuf[pl.ds(v * EMBED_DIM + LANES, LANES)]
                    )
        line0 = f * LINES_PER_FIELD + FULL_CHUNKS * BLK_LINES + BLK2_LINES
        pltpu.sync_copy(
            out0.at[pl.ds(0, TAIL_LINES), pl.ds(0, 128)],
            lines_out.at[pl.ds(line0, TAIL_LINES), :],
        )


def _gk_body(lines_hbm, x_hbm, out_hbm, xv, idxl, qv, linebuf, rows_v, sem):
    wid = _wid()
    b0 = wid * XB_PER_W
    iota = lax.broadcasted_iota(jnp.int32, (LANES,), 0)
    off_lo = iota * VOCAB_PAD
    off_hi = (iota + 10) * VOCAB_PAD

    def chunk(ci, carry):
        bl0 = ci * BC
        pltpu.sync_copy(x_hbm.at[pl.ds(b0 + bl0, BC), :], xv)

        def prep(b, cc):
            g1 = xv[b, pl.ds(0, LANES)] + off_lo
            g2 = xv[b, pl.ds(10, LANES)] + off_hi
            r = b * NUM_FIELDS
            idxl[pl.ds(r, LANES)] = lax.shift_right_logical(g1, 2)
            idxl[pl.ds(r + 10, LANES)] = lax.shift_right_logical(g2, 2)
            qv[pl.ds(r, LANES)] = lax.bitwise_and(g1, 3)
            qv[pl.ds(r + 10, LANES)] = lax.bitwise_and(g2, 3)
            return cc

        lax.fori_loop(0, BC, prep, 0, unroll=2)

        copies = []
        for j in range(N_SUB):
            copies.append(
                pltpu.async_copy(
                    lines_hbm.at[idxl.at[pl.ds(j * SUB, SUB)]],
                    linebuf.at[pl.ds(j * SUB, SUB)],
                    sem,
                )
            )
        for cp in copies:
            cp.wait()

        def extract(b, cc):
            r = b * NUM_FIELDS
            qa = qv[pl.ds(r, LANES)]
            qb = qv[pl.ds(r + 10, LANES)]
            for f in range(NUM_FIELDS):
                q32 = (qa[f] if f < LANES else qb[f - 10]) * EMBED_DIM
                rows_v[b, pl.ds(f * EMBED_DIM, LANES)] = (
                    linebuf[r + f, pl.ds(q32, LANES)]
                )
                rows_v[b, pl.ds(f * EMBED_DIM + LANES, LANES)] = (
                    linebuf[r + f, pl.ds(q32 + LANES, LANES)]
                )
            return cc

        lax.fori_loop(0, BC, extract, 0, unroll=1)

        pltpu.sync_copy(rows_v, out_hbm.at[pl.ds(b0 + bl0, BC), :])
        return carry

    lax.fori_loop(0, N_BC, chunk, 0)


@jax.jit
def kernel(x, tables):
    tab_t = jnp.swapaxes(tables, 1, 2)
    mesh = plsc.VectorSubcoreMesh(core_axis_name="c", subcore_axis_name="s")
    params = pltpu.CompilerParams(
        use_tc_tiling_on_sc=True, needs_layout_passes=False
    )

    tail33 = tables[:, FULL_CHUNKS * CV + CV2:, :].reshape(TAIL33_WORDS)

    lines = pl.kernel(
        _tr_body,
        out_type=jax.ShapeDtypeStruct((TOT_LINES, 128), jnp.float32),
        mesh=mesh,
        compiler_params=params,
        scratch_types=[
            pltpu.VMEM((EMBED_DIM, CV + 129), jnp.float32),
            pltpu.VMEM((EMBED_DIM, CV + 129), jnp.float32),
            pltpu.VMEM((BLK_LINES, 132), jnp.float32),
            pltpu.VMEM((BLK_LINES, 132), jnp.float32),
            pltpu.VMEM((TAIL33 * EMBED_DIM,), jnp.float32),
            pltpu.SemaphoreType.DMA,
            pltpu.SemaphoreType.DMA,
            pltpu.SemaphoreType.DMA,
            pltpu.SemaphoreType.DMA,
        ],
    )(tab_t, tail33)

    out = pl.kernel(
        _gk_body,
        out_type=jax.ShapeDtypeStruct((BATCH, ROW_W), jnp.float32),
        mesh=mesh,
        compiler_params=params,
        scratch_types=[
            pltpu.VMEM((BC, NUM_FIELDS), jnp.int32),
            pltpu.VMEM((CROWS,), jnp.int32),
            pltpu.VMEM((CROWS,), jnp.int32),
            pltpu.VMEM((CROWS, 128), jnp.float32),
            pltpu.VMEM((BC, ROW_W), jnp.float32),
            pltpu.SemaphoreType.DMA,
        ],
    )(lines, x)
    return out


# speedup vs baseline: 4.5452x; 1.1545x over previous
import jax
import jax.numpy as jnp
from jax import lax
from jax.experimental import pallas as pl
from jax.experimental.pallas import tpu as pltpu
from jax.experimental.pallas import tpu_sc as plsc

NUM_FIELDS = 26
VOCAB_P1 = 100001
EMBED_DIM = 32
BATCH = 16384
ROW_W = NUM_FIELDS * EMBED_DIM

NC = 2
NS = 16
LANES = 16
NW = NC * NS

VOCAB_PAD = 100032
LINES_PER_FIELD = VOCAB_PAD // 4
TOT_LINES = NUM_FIELDS * LINES_PER_FIELD

CV = 512
FULL_CHUNKS = VOCAB_P1 // CV
CV2 = 128
N_FULL = NUM_FIELDS * FULL_CHUNKS
K_MAX = (N_FULL + NW - 1) // NW
T_STEPS = (K_MAX + 2 + 1) // 2
BLK_LINES = CV * EMBED_DIM // 128
BLK2_LINES = CV2 * EMBED_DIM // 128
TAIL33 = VOCAB_P1 - FULL_CHUNKS * CV - CV2
TAIL33_WORDS = NUM_FIELDS * TAIL33 * EMBED_DIM
TAIL_LINES = (VOCAB_PAD - FULL_CHUNKS * CV - CV2) * EMBED_DIM // 128

XB_PER_W = BATCH // NW
BC = 8
N_BC = XB_PER_W // BC
CROWS = BC * NUM_FIELDS
SUB = 104
N_SUB = CROWS // SUB


def _wid():
    return lax.axis_index("s") * NC + lax.axis_index("c")


def _tr_body(tab_t, tail33, lines_out, in0, in1, out0, out1, tbuf,
             si0, si1, so0, so1):
    wid = _wid()
    iota = lax.broadcasted_iota(jnp.int32, (LANES,), 0)

    def src_of(item):
        f = item // FULL_CHUNKS
        ci = item % FULL_CHUNKS
        return f, ci, tab_t.at[f, :, pl.ds(ci * CV, CV)]

    def dst_of(item):
        f = item // FULL_CHUNKS
        ci = item % FULL_CHUNKS
        line0 = f * LINES_PER_FIELD + ci * BLK_LINES
        return lines_out.at[pl.ds(line0, BLK_LINES), :]

    rowpat = lax.shift_right_logical(iota, 2)
    colpat = lax.bitwise_and(iota, 3) * EMBED_DIM

    def transpose_block(inb, outb, n_v):
        @plsc.parallel_loop(0, n_v // LANES, 1, unroll=2)
        def _(vv):
            rowv = rowpat + vv * 4
            for d in range(EMBED_DIM):
                val = inb[d, pl.ds(vv * LANES, LANES)]
                plsc.store_scatter(outb, [rowv, colpat + d], val)

    bufs = ((in0, out0, si0, so0), (in1, out1, si1, so1))

    for par in range(2):
        item = par * NW + wid

        @pl.when(item < N_FULL)
        def _(par=par, item=item):
            inb, _, sin, _ = bufs[par]
            pltpu.async_copy(src_of(item)[2], inb.at[:, pl.ds(0, CV)], sin)

    def step(t, c):
        for par in range(2):
            k = t * 2 + par
            item = k * NW + wid
            inb, outb, sin, sout = bufs[par]

            @pl.when((k >= 2) & ((k - 2) * NW + wid < N_FULL))
            def _():
                pltpu.make_async_copy(outb.at[:, pl.ds(0, 128)], dst_of((k - 2) * NW + wid), sout).wait()

            @pl.when(item < N_FULL)
            def _():
                pltpu.make_async_copy(src_of(item)[2], inb.at[:, pl.ds(0, CV)], sin).wait()
                transpose_block(inb, outb, CV)
                pltpu.async_copy(outb.at[:, pl.ds(0, 128)], dst_of(item), sout)

                nxt = (k + 2) * NW + wid

                @pl.when(nxt < N_FULL)
                def _():
                    pltpu.async_copy(src_of(nxt)[2], inb.at[:, pl.ds(0, CV)], sin)

        return c

    lax.fori_loop(0, T_STEPS, step, 0)

    @pl.when(wid < NUM_FIELDS)
    def _():
        f = wid
        pltpu.sync_copy(
            tab_t.at[f, :, pl.ds(FULL_CHUNKS * CV, CV2)],
            in0.at[:, pl.ds(0, CV2)],
        )
        transpose_block(in0, out0, CV2)
        line0 = f * LINES_PER_FIELD + FULL_CHUNKS * BLK_LINES
        pltpu.sync_copy(
            out0.at[pl.ds(0, BLK2_LINES), pl.ds(0, 128)],
            lines_out.at[pl.ds(line0, BLK2_LINES), :],
        )

        pltpu.sync_copy(
            tail33.at[pl.ds(f * TAIL33 * EMBED_DIM, TAIL33 * EMBED_DIM)], tbuf
        )
        for l in range(TAIL_LINES):
            for qq in range(4):
                v = l * 4 + qq
                if v < TAIL33:
                    out0[l, pl.ds(qq * EMBED_DIM, LANES)] = (
                        tbuf[pl.ds(v * EMBED_DIM, LANES)]
                    )
                    out0[l, pl.ds(qq * EMBED_DIM + LANES, LANES)] = (
                        tbuf[pl.ds(v * EMBED_DIM + LANES, LANES)]
                    )
        line0 = f * LINES_PER_FIELD + FULL_CHUNKS * BLK_LINES + BLK2_LINES
        pltpu.sync_copy(
            out0.at[pl.ds(0, TAIL_LINES), pl.ds(0, 128)],
            lines_out.at[pl.ds(line0, TAIL_LINES), :],
        )


def _gk_body(lines_hbm, x_hbm, out_hbm, xv, idxl, qv, linebuf, rows_v, sem):
    wid = _wid()
    b0 = wid * XB_PER_W
    iota = lax.broadcasted_iota(jnp.int32, (LANES,), 0)
    off_lo = iota * VOCAB_PAD
    off_hi = (iota + 10) * VOCAB_PAD

    def chunk(ci, carry):
        bl0 = ci * BC
        pltpu.sync_copy(x_hbm.at[pl.ds(b0 + bl0, BC), :], xv)

        def prep(b, cc):
            g1 = xv[b, pl.ds(0, LANES)] + off_lo
            g2 = xv[b, pl.ds(10, LANES)] + off_hi
            r = b * NUM_FIELDS
            idxl[pl.ds(r, LANES)] = lax.shift_right_logical(g1, 2)
            idxl[pl.ds(r + 10, LANES)] = lax.shift_right_logical(g2, 2)
            qv[pl.ds(r, LANES)] = lax.bitwise_and(g1, 3)
            qv[pl.ds(r + 10, LANES)] = lax.bitwise_and(g2, 3)
            return cc

        lax.fori_loop(0, BC, prep, 0, unroll=2)

        copies = []
        for j in range(N_SUB):
            copies.append(
                pltpu.async_copy(
                    lines_hbm.at[idxl.at[pl.ds(j * SUB, SUB)]],
                    linebuf.at[pl.ds(j * SUB, SUB)],
                    sem,
                )
            )
        for cp in copies:
            cp.wait()

        def extract(b, cc):
            r = b * NUM_FIELDS
            qa = qv[pl.ds(r, LANES)]
            qb = qv[pl.ds(r + 10, LANES)]
            for f in range(NUM_FIELDS):
                q32 = (qa[f] if f < LANES else qb[f - 10]) * EMBED_DIM
                rows_v[b, pl.ds(f * EMBED_DIM, LANES)] = (
                    linebuf[r + f, pl.ds(q32, LANES)]
                )
                rows_v[b, pl.ds(f * EMBED_DIM + LANES, LANES)] = (
                    linebuf[r + f, pl.ds(q32 + LANES, LANES)]
                )
            return cc

        lax.fori_loop(0, BC, extract, 0, unroll=1)

        pltpu.sync_copy(rows_v, out_hbm.at[pl.ds(b0 + bl0, BC), :])
        return carry

    lax.fori_loop(0, N_BC, chunk, 0)


@jax.jit
def kernel(x, tables):
    tab_t = jnp.swapaxes(tables, 1, 2)
    mesh = plsc.VectorSubcoreMesh(core_axis_name="c", subcore_axis_name="s")
    params = pltpu.CompilerParams(
        use_tc_tiling_on_sc=True, needs_layout_passes=False
    )

    tail33 = tables[:, FULL_CHUNKS * CV + CV2:, :].reshape(TAIL33_WORDS)

    lines = pl.kernel(
        _tr_body,
        out_type=jax.ShapeDtypeStruct((TOT_LINES, 128), jnp.float32),
        mesh=mesh,
        compiler_params=params,
        scratch_types=[
            pltpu.VMEM((EMBED_DIM, CV + 129), jnp.float32),
            pltpu.VMEM((EMBED_DIM, CV + 129), jnp.float32),
            pltpu.VMEM((BLK_LINES, 132), jnp.float32),
            pltpu.VMEM((BLK_LINES, 132), jnp.float32),
            pltpu.VMEM((TAIL33 * EMBED_DIM,), jnp.float32),
            pltpu.SemaphoreType.DMA,
            pltpu.SemaphoreType.DMA,
            pltpu.SemaphoreType.DMA,
            pltpu.SemaphoreType.DMA,
        ],
    )(tab_t, tail33)

    out = pl.kernel(
        _gk_body,
        out_type=jax.ShapeDtypeStruct((BATCH, ROW_W), jnp.float32),
        mesh=mesh,
        compiler_params=params,
        scratch_types=[
            pltpu.VMEM((BC, NUM_FIELDS), jnp.int32),
            pltpu.VMEM((CROWS,), jnp.int32),
            pltpu.VMEM((CROWS,), jnp.int32),
            pltpu.VMEM((CROWS, 128), jnp.float32),
            pltpu.VMEM((BC, ROW_W), jnp.float32),
            pltpu.SemaphoreType.DMA,
        ],
    )(lines, x)
    return out


# speedup vs baseline: 4.8636x; 1.0701x over previous
import jax
import jax.numpy as jnp
from jax import lax
from jax.experimental import pallas as pl
from jax.experimental.pallas import tpu as pltpu
from jax.experimental.pallas import tpu_sc as plsc

NUM_FIELDS = 26
VOCAB_P1 = 100001
EMBED_DIM = 32
BATCH = 16384
ROW_W = NUM_FIELDS * EMBED_DIM

NC = 2
NS = 16
LANES = 16
NW = NC * NS

VOCAB_PAD = 100032
LINES_PER_FIELD = VOCAB_PAD // 4
TOT_LINES = NUM_FIELDS * LINES_PER_FIELD

CV = 512
FULL_CHUNKS = VOCAB_P1 // CV
CV2 = 128
N_FULL = NUM_FIELDS * FULL_CHUNKS
K_MAX = (N_FULL + NW - 1) // NW
T_STEPS = (K_MAX + 2 + 1) // 2
BLK_LINES = CV * EMBED_DIM // 128
BLK2_LINES = CV2 * EMBED_DIM // 128
TAIL33 = VOCAB_P1 - FULL_CHUNKS * CV - CV2
TAIL33_WORDS = NUM_FIELDS * TAIL33 * EMBED_DIM
TAIL_LINES = (VOCAB_PAD - FULL_CHUNKS * CV - CV2) * EMBED_DIM // 128

XB_PER_W = BATCH // NW
BC = 8
N_BC = XB_PER_W // BC
CROWS = BC * NUM_FIELDS
SUB = 104
N_SUB = CROWS // SUB


def _wid():
    return lax.axis_index("s") * NC + lax.axis_index("c")


def _tr_body(tab_t, tail33, lines_out, in0, in1, out0, out1, tbuf,
             si0, si1, so0, so1):
    wid = _wid()
    iota = lax.broadcasted_iota(jnp.int32, (LANES,), 0)

    def src_of(item):
        f = item // FULL_CHUNKS
        ci = item % FULL_CHUNKS
        return f, ci, tab_t.at[f, :, pl.ds(ci * CV, CV)]

    def dst_of(item):
        f = item // FULL_CHUNKS
        ci = item % FULL_CHUNKS
        line0 = f * LINES_PER_FIELD + ci * BLK_LINES
        return lines_out.at[pl.ds(line0, BLK_LINES), :]

    rowpat = lax.shift_right_logical(iota, 2)
    colpat = lax.bitwise_and(iota, 3) * EMBED_DIM

    def transpose_block(inb, outb, n_v):
        @plsc.parallel_loop(0, n_v // LANES, 1, unroll=4)
        def _(vv):
            rowv = rowpat + vv * 4
            for d in range(EMBED_DIM):
                val = inb[d, pl.ds(vv * LANES, LANES)]
                plsc.store_scatter(outb, [rowv, colpat + d], val)

    bufs = ((in0, out0, si0, so0), (in1, out1, si1, so1))

    for par in range(2):
        item = par * NW + wid

        @pl.when(item < N_FULL)
        def _(par=par, item=item):
            inb, _, sin, _ = bufs[par]
            pltpu.async_copy(src_of(item)[2], inb.at[:, pl.ds(0, CV)], sin)

    def step(t, c):
        for par in range(2):
            k = t * 2 + par
            item = k * NW + wid
            inb, outb, sin, sout = bufs[par]

            @pl.when((k >= 2) & ((k - 2) * NW + wid < N_FULL))
            def _():
                pltpu.make_async_copy(outb.at[:, pl.ds(0, 128)], dst_of((k - 2) * NW + wid), sout).wait()

            @pl.when(item < N_FULL)
            def _():
                pltpu.make_async_copy(src_of(item)[2], inb.at[:, pl.ds(0, CV)], sin).wait()
                transpose_block(inb, outb, CV)
                pltpu.async_copy(outb.at[:, pl.ds(0, 128)], dst_of(item), sout)

                nxt = (k + 2) * NW + wid

                @pl.when(nxt < N_FULL)
                def _():
                    pltpu.async_copy(src_of(nxt)[2], inb.at[:, pl.ds(0, CV)], sin)

        return c

    lax.fori_loop(0, T_STEPS, step, 0)

    @pl.when(wid < NUM_FIELDS)
    def _():
        f = wid
        pltpu.sync_copy(
            tab_t.at[f, :, pl.ds(FULL_CHUNKS * CV, CV2)],
            in0.at[:, pl.ds(0, CV2)],
        )
        transpose_block(in0, out0, CV2)
        line0 = f * LINES_PER_FIELD + FULL_CHUNKS * BLK_LINES
        pltpu.sync_copy(
            out0.at[pl.ds(0, BLK2_LINES), pl.ds(0, 128)],
            lines_out.at[pl.ds(line0, BLK2_LINES), :],
        )

        pltpu.sync_copy(
            tail33.at[pl.ds(f * TAIL33 * EMBED_DIM, TAIL33 * EMBED_DIM)], tbuf
        )
        for l in range(TAIL_LINES):
            for qq in range(4):
                v = l * 4 + qq
                if v < TAIL33:
                    out0[l, pl.ds(qq * EMBED_DIM, LANES)] = (
                        tbuf[pl.ds(v * EMBED_DIM, LANES)]
                    )
                    out0[l, pl.ds(qq * EMBED_DIM + LANES, LANES)] = (
                        tbuf[pl.ds(v * EMBED_DIM + LANES, LANES)]
                    )
        line0 = f * LINES_PER_FIELD + FULL_CHUNKS * BLK_LINES + BLK2_LINES
        pltpu.sync_copy(
            out0.at[pl.ds(0, TAIL_LINES), pl.ds(0, 128)],
            lines_out.at[pl.ds(line0, TAIL_LINES), :],
        )


def _gk_body(lines_hbm, x_hbm, out_hbm, xv, idxl, qv, linebuf, rows_v, sem):
    wid = _wid()
    b0 = wid * XB_PER_W
    iota = lax.broadcasted_iota(jnp.int32, (LANES,), 0)
    off_lo = iota * VOCAB_PAD
    off_hi = (iota + 10) * VOCAB_PAD

    def chunk(ci, carry):
        bl0 = ci * BC
        pltpu.sync_copy(x_hbm.at[pl.ds(b0 + bl0, BC), :], xv)

        @plsc.parallel_loop(0, BC, 1, unroll=2)
        def _(b):
            g1 = xv[b, pl.ds(0, LANES)] + off_lo
            g2 = xv[b, pl.ds(10, LANES)] + off_hi
            r = b * NUM_FIELDS
            idxl[pl.ds(r, LANES)] = lax.shift_right_logical(g1, 2)
            idxl[pl.ds(r + 10, LANES)] = lax.shift_right_logical(g2, 2)
            qv[pl.ds(r, LANES)] = lax.bitwise_and(g1, 3)
            qv[pl.ds(r + 10, LANES)] = lax.bitwise_and(g2, 3)

        copies = []
        for j in range(N_SUB):
            copies.append(
                pltpu.async_copy(
                    lines_hbm.at[idxl.at[pl.ds(j * SUB, SUB)]],
                    linebuf.at[pl.ds(j * SUB, SUB)],
                    sem,
                )
            )
        for cp in copies:
            cp.wait()

        @plsc.parallel_loop(0, BC, 1, unroll=2)
        def _(b):
            r = b * NUM_FIELDS
            qa = qv[pl.ds(r, LANES)]
            qb = qv[pl.ds(r + 10, LANES)]
            for f in range(NUM_FIELDS):
                q32 = (qa[f] if f < LANES else qb[f - 10]) * EMBED_DIM
                rows_v[b, pl.ds(f * EMBED_DIM, LANES)] = (
                    linebuf[r + f, pl.ds(q32, LANES)]
                )
                rows_v[b, pl.ds(f * EMBED_DIM + LANES, LANES)] = (
                    linebuf[r + f, pl.ds(q32 + LANES, LANES)]
                )

        pltpu.sync_copy(rows_v, out_hbm.at[pl.ds(b0 + bl0, BC), :])
        return carry

    lax.fori_loop(0, N_BC, chunk, 0)


@jax.jit
def kernel(x, tables):
    tab_t = jnp.swapaxes(tables, 1, 2)
    mesh = plsc.VectorSubcoreMesh(core_axis_name="c", subcore_axis_name="s")
    params = pltpu.CompilerParams(
        use_tc_tiling_on_sc=True, needs_layout_passes=False
    )

    tail33 = tables[:, FULL_CHUNKS * CV + CV2:, :].reshape(TAIL33_WORDS)

    lines = pl.kernel(
        _tr_body,
        out_type=jax.ShapeDtypeStruct((TOT_LINES, 128), jnp.float32),
        mesh=mesh,
        compiler_params=params,
        scratch_types=[
            pltpu.VMEM((EMBED_DIM, CV + 129), jnp.float32),
            pltpu.VMEM((EMBED_DIM, CV + 129), jnp.float32),
            pltpu.VMEM((BLK_LINES, 132), jnp.float32),
            pltpu.VMEM((BLK_LINES, 132), jnp.float32),
            pltpu.VMEM((TAIL33 * EMBED_DIM,), jnp.float32),
            pltpu.SemaphoreType.DMA,
            pltpu.SemaphoreType.DMA,
            pltpu.SemaphoreType.DMA,
            pltpu.SemaphoreType.DMA,
        ],
    )(tab_t, tail33)

    out = pl.kernel(
        _gk_body,
        out_type=jax.ShapeDtypeStruct((BATCH, ROW_W), jnp.float32),
        mesh=mesh,
        compiler_params=params,
        scratch_types=[
            pltpu.VMEM((BC, NUM_FIELDS), jnp.int32),
            pltpu.VMEM((CROWS,), jnp.int32),
            pltpu.VMEM((CROWS,), jnp.int32),
            pltpu.VMEM((CROWS, 128), jnp.float32),
            pltpu.VMEM((BC, ROW_W), jnp.float32),
            pltpu.SemaphoreType.DMA,
        ],
    )(lines, x)
    return out
